# Initial kernel scaffold; baseline (speedup 1.0000x reference)
import jax
import jax.numpy as jnp
from jax.experimental import pallas as pl


def kernel(node_features, edge_features, pair_indices, kernel, bias):
    raise NotImplementedError("write your pallas kernel here")


# speedup vs baseline: 2.2883x; 2.2883x over previous
import functools

import jax
import jax.numpy as jnp
from jax import lax
from jax.experimental import pallas as pl
from jax.experimental.pallas import tpu as pltpu
from jax.experimental.pallas import tpu_sc as plsc

N = 10000
E = 160000
NFC = 32
EFC = 16

NC = 2
NS = 16
NW = NC * NS

CH = 128
MAIN_CHUNKS = (E // CH) // NW
PER_W = MAIN_CHUNKS * CH
TAIL_BASE = PER_W * NW
TAIL_WORKERS = (E - TAIL_BASE) // CH

B_E = 640


def _worker_id():
    return lax.axis_index("s") * NC + lax.axis_index("c")


def _copy_idx_window(idx_all, idx_chunk, off):
    for k in range(CH // 16):
        idx_chunk[pl.ds(k * 16, 16)] = idx_all[pl.ds(off + k * 16, 16)]


def _chunk_offsets(wid, i):
    base = wid * PER_W
    main_off = base + i * CH
    tail_off = TAIL_BASE + wid * CH
    is_tail = i >= MAIN_CHUNKS
    g_off = jnp.where(is_tail, tail_off, main_off)
    l_off = jnp.where(is_tail, PER_W, i * CH)
    return g_off, l_off


def _make_gather():
    mesh = plsc.VectorSubcoreMesh(core_axis_name="c", subcore_axis_name="s")

    @functools.partial(
        pl.kernel,
        mesh=mesh,
        out_type=jax.ShapeDtypeStruct((E, NFC), jnp.float32),
        scratch_types=[
            pltpu.VMEM((PER_W + CH,), jnp.int32),
            pltpu.VMEM((CH,), jnp.int32),
            pltpu.VMEM((CH, NFC), jnp.float32),
            pltpu.SemaphoreType.DMA,
        ],
        compiler_params=pltpu.CompilerParams(use_tc_tiling_on_sc=False),
    )
    def gather_k(table_hbm, idx_hbm, out_hbm, idx_all, idx_chunk, rows_v, sem):
        wid = _worker_id()
        base = wid * PER_W
        pltpu.sync_copy(idx_hbm.at[pl.ds(base, PER_W)], idx_all.at[pl.ds(0, PER_W)])

        @pl.when(wid < TAIL_WORKERS)
        def _():
            pltpu.sync_copy(
                idx_hbm.at[pl.ds(TAIL_BASE + wid * CH, CH)],
                idx_all.at[pl.ds(PER_W, CH)],
            )

        n_my = MAIN_CHUNKS + jnp.where(wid < TAIL_WORKERS, 1, 0)

        def body(i, _):
            g_off, l_off = _chunk_offsets(wid, i)
            _copy_idx_window(idx_all, idx_chunk, l_off)
            pltpu.async_copy(table_hbm.at[idx_chunk], rows_v, sem).wait()
            pltpu.sync_copy(rows_v, out_hbm.at[pl.ds(g_off, CH)])
            return 0

        lax.fori_loop(0, n_my, body, 0)

    return gather_k


def _make_scatter():
    mesh = plsc.VectorSubcoreMesh(core_axis_name="c", subcore_axis_name="s")
    rows_per_sub = N // NS

    @functools.partial(
        pl.kernel,
        mesh=mesh,
        out_type=jax.ShapeDtypeStruct((NC, N, NFC), jnp.float32),
        scratch_types=[
            pltpu.VMEM((PER_W + CH,), jnp.int32),
            pltpu.VMEM((CH,), jnp.int32),
            pltpu.VMEM((CH, NFC), jnp.float32),
            pltpu.VMEM_SHARED((N, NFC), jnp.float32),
        ],
        compiler_params=pltpu.CompilerParams(use_tc_tiling_on_sc=False),
    )
    def scatter_k(rows_hbm, idx_hbm, zeros_hbm, out_hbm, idx_all, idx_chunk, rows_v, acc):
        cid = lax.axis_index("c")
        sid = lax.axis_index("s")
        wid = _worker_id()
        base = wid * PER_W

        pltpu.sync_copy(
            zeros_hbm.at[pl.ds(sid * rows_per_sub, rows_per_sub)],
            acc.at[pl.ds(sid * rows_per_sub, rows_per_sub)],
        )
        pltpu.sync_copy(idx_hbm.at[pl.ds(base, PER_W)], idx_all.at[pl.ds(0, PER_W)])

        @pl.when(wid < TAIL_WORKERS)
        def _():
            pltpu.sync_copy(
                idx_hbm.at[pl.ds(TAIL_BASE + wid * CH, CH)],
                idx_all.at[pl.ds(PER_W, CH)],
            )

        plsc.subcore_barrier()

        n_my = MAIN_CHUNKS + jnp.where(wid < TAIL_WORKERS, 1, 0)

        def body(i, _):
            g_off, l_off = _chunk_offsets(wid, i)
            _copy_idx_window(idx_all, idx_chunk, l_off)
            pltpu.sync_copy(rows_hbm.at[pl.ds(g_off, CH)], rows_v)
            pltpu.sync_copy(rows_v, acc.at[idx_chunk], add=True)
            return 0

        lax.fori_loop(0, n_my, body, 0)

        plsc.subcore_barrier()
        pltpu.sync_copy(
            acc.at[pl.ds(sid * rows_per_sub, rows_per_sub)],
            out_hbm.at[cid, pl.ds(sid * rows_per_sub, rows_per_sub)],
        )

    return scatter_k


def _transform_body(edge_ref, neigh_ref, w_ref, bt_ref, out_ref):
    e = edge_ref[...]
    n = neigh_ref[...]
    p = jnp.concatenate(
        [e[:, c:c + 1] * n for c in range(EFC)], axis=1
    )
    out = lax.dot_general(
        p, w_ref[...], (((1,), (0,)), ((), ())),
        preferred_element_type=jnp.float32,
    )
    out += lax.dot_general(
        n, bt_ref[...], (((1,), (0,)), ((), ())),
        preferred_element_type=jnp.float32,
    )
    out_ref[...] = out


def _transform(edge_features, neigh, w, bt):
    grid = E // B_E
    return pl.pallas_call(
        _transform_body,
        grid=(grid,),
        in_specs=[
            pl.BlockSpec((B_E, EFC), lambda i: (i, 0)),
            pl.BlockSpec((B_E, NFC), lambda i: (i, 0)),
            pl.BlockSpec((EFC * NFC, NFC), lambda i: (0, 0)),
            pl.BlockSpec((NFC, NFC), lambda i: (0, 0)),
        ],
        out_specs=pl.BlockSpec((B_E, NFC), lambda i: (i, 0)),
        out_shape=jax.ShapeDtypeStruct((E, NFC), jnp.float32),
        compiler_params=pltpu.CompilerParams(
            dimension_semantics=("arbitrary",),
        ),
    )(edge_features, neigh, w, bt)


def _combine_body(a_ref, b_ref, out_ref):
    out_ref[...] = a_ref[...] + b_ref[...]


def _combine(partials):
    return pl.pallas_call(
        _combine_body,
        out_shape=jax.ShapeDtypeStruct((N, NFC), jnp.float32),
    )(partials[0], partials[1])


@functools.cache
def _sc_calls():
    return _make_gather(), _make_scatter()


def kernel(node_features, edge_features, pair_indices, kernel, bias):
    _gather_call, _scatter_call = _sc_calls()
    idx_dst = pair_indices[:, 0].astype(jnp.int32)
    idx_src = pair_indices[:, 1].astype(jnp.int32)
    w = kernel.reshape(EFC, NFC, NFC).transpose(0, 2, 1).reshape(EFC * NFC, NFC)
    bt = bias.reshape(NFC, NFC).T
    zeros = jnp.zeros((N, NFC), jnp.float32)

    neigh = _gather_call(node_features, idx_src)
    transformed = _transform(edge_features, neigh, w, bt)
    partials = _scatter_call(transformed, idx_dst, zeros)
    return _combine(partials)


# speedup vs baseline: 3.7216x; 1.6264x over previous
import functools

import jax
import jax.numpy as jnp
from jax import lax
from jax.experimental import pallas as pl
from jax.experimental.pallas import tpu as pltpu
from jax.experimental.pallas import tpu_sc as plsc

N = 10000
E = 160000
NFC = 32
EFC = 16

NC = 2
NS = 16
NW = NC * NS

CH = 128
MAIN_CHUNKS = (E // CH) // NW
PER_W = MAIN_CHUNKS * CH
TAIL_BASE = PER_W * NW
TAIL_WORKERS = (E - TAIL_BASE) // CH

B_E = 640


def _worker_id():
    return lax.axis_index("s") * NC + lax.axis_index("c")


def _copy_idx_window(idx_all, idx_chunk, off):
    for k in range(CH // 16):
        idx_chunk[pl.ds(k * 16, 16)] = idx_all[pl.ds(off + k * 16, 16)]


def _chunk_offsets(wid, i):
    base = wid * PER_W
    main_off = base + i * CH
    tail_off = TAIL_BASE + wid * CH
    is_tail = i >= MAIN_CHUNKS
    g_off = jnp.where(is_tail, tail_off, main_off)
    l_off = jnp.where(is_tail, PER_W, i * CH)
    return g_off, l_off


def _make_gather():
    mesh = plsc.VectorSubcoreMesh(core_axis_name="c", subcore_axis_name="s")

    @functools.partial(
        pl.kernel,
        mesh=mesh,
        out_type=jax.ShapeDtypeStruct((E, NFC), jnp.float32),
        scratch_types=[
            pltpu.VMEM((PER_W + CH,), jnp.int32),
            pltpu.VMEM((CH,), jnp.int32),
            pltpu.VMEM((CH, NFC), jnp.float32),
            pltpu.SemaphoreType.DMA,
        ],
        compiler_params=pltpu.CompilerParams(use_tc_tiling_on_sc=False),
    )
    def gather_k(table_hbm, idx_hbm, out_hbm, idx_all, idx_chunk, rows_v, sem):
        wid = _worker_id()
        base = wid * PER_W
        pltpu.sync_copy(idx_hbm.at[pl.ds(base, PER_W)], idx_all.at[pl.ds(0, PER_W)])

        @pl.when(wid < TAIL_WORKERS)
        def _():
            pltpu.sync_copy(
                idx_hbm.at[pl.ds(TAIL_BASE + wid * CH, CH)],
                idx_all.at[pl.ds(PER_W, CH)],
            )

        n_my = MAIN_CHUNKS + jnp.where(wid < TAIL_WORKERS, 1, 0)

        def body(i, _):
            g_off, l_off = _chunk_offsets(wid, i)
            _copy_idx_window(idx_all, idx_chunk, l_off)
            pltpu.async_copy(table_hbm.at[idx_chunk], rows_v, sem).wait()
            pltpu.sync_copy(rows_v, out_hbm.at[pl.ds(g_off, CH)])
            return 0

        lax.fori_loop(0, n_my, body, 0)

    return gather_k


def _make_scatter():
    mesh = plsc.VectorSubcoreMesh(core_axis_name="c", subcore_axis_name="s")
    rows_per_sub = N // NS

    @functools.partial(
        pl.kernel,
        mesh=mesh,
        out_type=jax.ShapeDtypeStruct((NC, N, NFC), jnp.float32),
        scratch_types=[
            pltpu.VMEM((PER_W + CH,), jnp.int32),
            pltpu.VMEM((CH,), jnp.int32),
            pltpu.VMEM((CH, NFC), jnp.float32),
            pltpu.VMEM_SHARED((N, NFC), jnp.float32),
        ],
        compiler_params=pltpu.CompilerParams(use_tc_tiling_on_sc=False),
    )
    def scatter_k(rows_hbm, idx_hbm, zeros_hbm, out_hbm, idx_all, idx_chunk, rows_v, acc):
        cid = lax.axis_index("c")
        sid = lax.axis_index("s")
        wid = _worker_id()
        base = wid * PER_W

        pltpu.sync_copy(
            zeros_hbm.at[pl.ds(sid * rows_per_sub, rows_per_sub)],
            acc.at[pl.ds(sid * rows_per_sub, rows_per_sub)],
        )
        pltpu.sync_copy(idx_hbm.at[pl.ds(base, PER_W)], idx_all.at[pl.ds(0, PER_W)])

        @pl.when(wid < TAIL_WORKERS)
        def _():
            pltpu.sync_copy(
                idx_hbm.at[pl.ds(TAIL_BASE + wid * CH, CH)],
                idx_all.at[pl.ds(PER_W, CH)],
            )

        plsc.subcore_barrier()

        n_my = MAIN_CHUNKS + jnp.where(wid < TAIL_WORKERS, 1, 0)

        def body(i, _):
            g_off, l_off = _chunk_offsets(wid, i)
            _copy_idx_window(idx_all, idx_chunk, l_off)
            pltpu.sync_copy(rows_hbm.at[pl.ds(g_off, CH)], rows_v)
            pltpu.sync_copy(rows_v, acc.at[idx_chunk], add=True)
            return 0

        lax.fori_loop(0, n_my, body, 0)

        plsc.subcore_barrier()
        pltpu.sync_copy(
            acc.at[pl.ds(sid * rows_per_sub, rows_per_sub)],
            out_hbm.at[cid, pl.ds(sid * rows_per_sub, rows_per_sub)],
        )

    return scatter_k


def _dot(a, b):
    return lax.dot_general(
        a, b, (((1,), (0,)), ((), ())), preferred_element_type=jnp.float32
    )


def _transform_body(edge_ref, neigh_ref, s_ref, t_ref, w_ref, bt_ref, out_ref):
    e = edge_ref[...]
    n = neigh_ref[...]
    erep = _dot(e, s_ref[...])
    ntile = _dot(n, t_ref[...])
    p = erep * ntile
    out_ref[...] = _dot(p, w_ref[...]) + _dot(n, bt_ref[...])


def _transform(edge_features, neigh, w, bt, s, t):
    grid = E // B_E
    return pl.pallas_call(
        _transform_body,
        grid=(grid,),
        in_specs=[
            pl.BlockSpec((B_E, EFC), lambda i: (i, 0)),
            pl.BlockSpec((B_E, NFC), lambda i: (i, 0)),
            pl.BlockSpec((EFC, EFC * NFC), lambda i: (0, 0)),
            pl.BlockSpec((NFC, EFC * NFC), lambda i: (0, 0)),
            pl.BlockSpec((EFC * NFC, NFC), lambda i: (0, 0)),
            pl.BlockSpec((NFC, NFC), lambda i: (0, 0)),
        ],
        out_specs=pl.BlockSpec((B_E, NFC), lambda i: (i, 0)),
        out_shape=jax.ShapeDtypeStruct((E, NFC), jnp.float32),
        compiler_params=pltpu.CompilerParams(
            dimension_semantics=("arbitrary",),
        ),
    )(edge_features, neigh, s, t, w, bt)


def _combine_body(a_ref, b_ref, out_ref):
    out_ref[...] = a_ref[...] + b_ref[...]


def _combine(partials):
    return pl.pallas_call(
        _combine_body,
        out_shape=jax.ShapeDtypeStruct((N, NFC), jnp.float32),
    )(partials[0], partials[1])


@functools.cache
def _sc_calls():
    return _make_gather(), _make_scatter()


def kernel(node_features, edge_features, pair_indices, kernel, bias):
    _gather_call, _scatter_call = _sc_calls()
    idx_dst = pair_indices[:, 0].astype(jnp.int32)
    idx_src = pair_indices[:, 1].astype(jnp.int32)
    w = kernel.reshape(EFC, NFC, NFC).transpose(0, 2, 1).reshape(EFC * NFC, NFC)
    bt = bias.reshape(NFC, NFC).T
    zeros = jnp.zeros((N, NFC), jnp.float32)
    s = jnp.kron(jnp.eye(EFC, dtype=jnp.float32), jnp.ones((1, NFC), jnp.float32))
    t = jnp.tile(jnp.eye(NFC, dtype=jnp.float32), (1, EFC))

    neigh = _gather_call(node_features, idx_src)
    transformed = _transform(edge_features, neigh, w, bt, s, t)
    partials = _scatter_call(transformed, idx_dst, zeros)
    return _combine(partials)


# speedup vs baseline: 4.7292x; 1.2707x over previous
import functools

import jax
import jax.numpy as jnp
from jax import lax
from jax.experimental import pallas as pl
from jax.experimental.pallas import tpu as pltpu
from jax.experimental.pallas import tpu_sc as plsc

N = 10000
E = 160000
NFC = 32
EFC = 16

NC = 2
NS = 16
NW = NC * NS

CH = 128
MAIN_CHUNKS = (E // CH) // NW
PER_W = MAIN_CHUNKS * CH
TAIL_BASE = PER_W * NW
TAIL_WORKERS = (E - TAIL_BASE) // CH

B_E = 640


def _worker_id():
    return lax.axis_index("s") * NC + lax.axis_index("c")


def _copy_idx_window(idx_all, idx_chunk, off):
    for k in range(CH // 16):
        idx_chunk[pl.ds(k * 16, 16)] = idx_all[pl.ds(off + k * 16, 16)]


def _chunk_offsets(wid, i):
    base = wid * PER_W
    main_off = base + i * CH
    tail_off = TAIL_BASE + wid * CH
    is_tail = i >= MAIN_CHUNKS
    g_off = jnp.where(is_tail, tail_off, main_off)
    l_off = jnp.where(is_tail, PER_W, i * CH)
    return g_off, l_off


def _make_gather():
    mesh = plsc.VectorSubcoreMesh(core_axis_name="c", subcore_axis_name="s")

    @functools.partial(
        pl.kernel,
        mesh=mesh,
        out_type=jax.ShapeDtypeStruct((E, NFC), jnp.float32),
        scratch_types=[
            pltpu.VMEM((PER_W + CH,), jnp.int32),
            pltpu.VMEM((CH,), jnp.int32),
            pltpu.VMEM((CH, NFC), jnp.float32),
            pltpu.SemaphoreType.DMA,
        ],
        compiler_params=pltpu.CompilerParams(use_tc_tiling_on_sc=False),
    )
    def gather_k(table_hbm, idx_hbm, out_hbm, idx_all, idx_chunk, rows_v, sem):
        wid = _worker_id()
        base = wid * PER_W
        pltpu.sync_copy(idx_hbm.at[pl.ds(base, PER_W)], idx_all.at[pl.ds(0, PER_W)])

        @pl.when(wid < TAIL_WORKERS)
        def _():
            pltpu.sync_copy(
                idx_hbm.at[pl.ds(TAIL_BASE + wid * CH, CH)],
                idx_all.at[pl.ds(PER_W, CH)],
            )

        n_my = MAIN_CHUNKS + jnp.where(wid < TAIL_WORKERS, 1, 0)

        def body(i, _):
            g_off, l_off = _chunk_offsets(wid, i)
            _copy_idx_window(idx_all, idx_chunk, l_off)
            pltpu.async_copy(table_hbm.at[idx_chunk], rows_v, sem).wait()
            pltpu.sync_copy(rows_v, out_hbm.at[pl.ds(g_off, CH)])
            return 0

        lax.fori_loop(0, n_my, body, 0)

    return gather_k


def _make_scatter():
    mesh = plsc.VectorSubcoreMesh(core_axis_name="c", subcore_axis_name="s")
    rows_per_sub = N // NS

    @functools.partial(
        pl.kernel,
        mesh=mesh,
        out_type=jax.ShapeDtypeStruct((NC, N, NFC), jnp.float32),
        scratch_types=[
            pltpu.VMEM((PER_W + CH,), jnp.int32),
            pltpu.VMEM((CH,), jnp.int32),
            pltpu.VMEM((CH, NFC), jnp.float32),
            pltpu.VMEM_SHARED((N, NFC), jnp.float32),
        ],
        compiler_params=pltpu.CompilerParams(use_tc_tiling_on_sc=False),
    )
    def scatter_k(rows_hbm, idx_hbm, zeros_hbm, out_hbm, idx_all, idx_chunk, rows_v, acc):
        cid = lax.axis_index("c")
        sid = lax.axis_index("s")
        wid = _worker_id()
        base = wid * PER_W

        pltpu.sync_copy(
            zeros_hbm.at[pl.ds(sid * rows_per_sub, rows_per_sub)],
            acc.at[pl.ds(sid * rows_per_sub, rows_per_sub)],
        )
        pltpu.sync_copy(idx_hbm.at[pl.ds(base, PER_W)], idx_all.at[pl.ds(0, PER_W)])

        @pl.when(wid < TAIL_WORKERS)
        def _():
            pltpu.sync_copy(
                idx_hbm.at[pl.ds(TAIL_BASE + wid * CH, CH)],
                idx_all.at[pl.ds(PER_W, CH)],
            )

        plsc.subcore_barrier()

        n_my = MAIN_CHUNKS + jnp.where(wid < TAIL_WORKERS, 1, 0)

        def body(i, _):
            g_off, l_off = _chunk_offsets(wid, i)
            _copy_idx_window(idx_all, idx_chunk, l_off)
            pltpu.sync_copy(rows_hbm.at[pl.ds(g_off, CH)], rows_v)
            pltpu.sync_copy(rows_v, acc.at[idx_chunk], add=True)
            return 0

        lax.fori_loop(0, n_my, body, 0)

        plsc.subcore_barrier()
        pltpu.sync_copy(
            acc.at[pl.ds(sid * rows_per_sub, rows_per_sub)],
            out_hbm.at[cid, pl.ds(sid * rows_per_sub, rows_per_sub)],
        )

    return scatter_k


def _dot(a, b):
    return lax.dot_general(
        a, b, (((1,), (0,)), ((), ())), preferred_element_type=jnp.float32
    )


PK = 4
EP = E // PK
B_P = B_E // PK
KP = PK * EFC * NFC


def _transform_body(edge_ref, neigh_ref, s_ref, t_ref, w_ref, bt_ref, out_ref):
    e = edge_ref[...]
    n = neigh_ref[...]
    erep = _dot(e, s_ref[...])
    ntile = _dot(n, t_ref[...])
    p = erep * ntile
    out_ref[...] = _dot(p, w_ref[...]) + _dot(n, bt_ref[...])


def _transform(epack, npack, w, bt, s, t):
    grid = EP // B_P
    return pl.pallas_call(
        _transform_body,
        grid=(grid,),
        in_specs=[
            pl.BlockSpec((B_P, PK * EFC), lambda i: (i, 0)),
            pl.BlockSpec((B_P, PK * NFC), lambda i: (i, 0)),
            pl.BlockSpec((PK * EFC, KP), lambda i: (0, 0)),
            pl.BlockSpec((PK * NFC, KP), lambda i: (0, 0)),
            pl.BlockSpec((KP, PK * NFC), lambda i: (0, 0)),
            pl.BlockSpec((PK * NFC, PK * NFC), lambda i: (0, 0)),
        ],
        out_specs=pl.BlockSpec((B_P, PK * NFC), lambda i: (i, 0)),
        out_shape=jax.ShapeDtypeStruct((EP, PK * NFC), jnp.float32),
        compiler_params=pltpu.CompilerParams(
            dimension_semantics=("arbitrary",),
        ),
    )(epack, npack, s, t, w, bt)


def _combine_body(a_ref, b_ref, out_ref):
    out_ref[...] = a_ref[...] + b_ref[...]


def _combine(partials):
    np_rows = N * NFC // 128
    pp = partials.reshape(2, np_rows, 128)
    a, b = pp[0], pp[1]
    out = pl.pallas_call(
        _combine_body,
        out_shape=jax.ShapeDtypeStruct((np_rows, 128), jnp.float32),
    )(a, b)
    return out.reshape(N, NFC)


@functools.cache
def _sc_calls():
    return _make_gather(), _make_scatter()


def kernel(node_features, edge_features, pair_indices, kernel, bias):
    _gather_call, _scatter_call = _sc_calls()
    idx_dst = pair_indices[:, 0].astype(jnp.int32)
    idx_src = pair_indices[:, 1].astype(jnp.int32)
    w = kernel.reshape(EFC, NFC, NFC).transpose(0, 2, 1).reshape(EFC * NFC, NFC)
    bt = bias.reshape(NFC, NFC).T
    zeros = jnp.zeros((N, NFC), jnp.float32)
    eye4 = jnp.eye(PK, dtype=jnp.float32)
    s = jnp.kron(jnp.eye(EFC, dtype=jnp.float32), jnp.ones((1, NFC), jnp.float32))
    t = jnp.tile(jnp.eye(NFC, dtype=jnp.float32), (1, EFC))
    s4 = jnp.kron(eye4, s)
    t4 = jnp.kron(eye4, t)
    w4 = jnp.kron(eye4, w)
    bt4 = jnp.kron(eye4, bt)

    neigh = _gather_call(node_features, idx_src)
    epack = edge_features.reshape(EP, PK * EFC)
    npack = neigh.reshape(EP, PK * NFC)
    transformed = _transform(epack, npack, w4, bt4, s4, t4)
    partials = _scatter_call(transformed.reshape(E, NFC), idx_dst, zeros)
    return _combine(partials)


# speedup vs baseline: 5.0931x; 1.0770x over previous
import functools

import jax
import jax.numpy as jnp
from jax import lax
from jax.experimental import pallas as pl
from jax.experimental.pallas import tpu as pltpu
from jax.experimental.pallas import tpu_sc as plsc

N = 10000
E = 160000
NFC = 32
EFC = 16

NC = 2
NS = 16
NW = NC * NS

CH = 128
MAIN_CHUNKS = (E // CH) // NW
PER_W = MAIN_CHUNKS * CH
TAIL_BASE = PER_W * NW
TAIL_WORKERS = (E - TAIL_BASE) // CH

B_E = 640


def _worker_id():
    return lax.axis_index("s") * NC + lax.axis_index("c")


def _copy_idx_window(idx_all, idx_chunk, off):
    for k in range(CH // 16):
        idx_chunk[pl.ds(k * 16, 16)] = idx_all[pl.ds(off + k * 16, 16)]


def _chunk_offsets(wid, i):
    base = wid * PER_W
    main_off = base + i * CH
    tail_off = TAIL_BASE + wid * CH
    is_tail = i >= MAIN_CHUNKS
    g_off = jnp.where(is_tail, tail_off, main_off)
    l_off = jnp.where(is_tail, PER_W, i * CH)
    return g_off, l_off


def _make_gather():
    mesh = plsc.VectorSubcoreMesh(core_axis_name="c", subcore_axis_name="s")

    @functools.partial(
        pl.kernel,
        mesh=mesh,
        out_type=jax.ShapeDtypeStruct((E, NFC), jnp.float32),
        scratch_types=[
            pltpu.VMEM((PER_W + CH,), jnp.int32),
            pltpu.VMEM((CH,), jnp.int32),
            pltpu.VMEM((CH, NFC), jnp.float32),
            pltpu.SemaphoreType.DMA,
        ],
        compiler_params=pltpu.CompilerParams(use_tc_tiling_on_sc=False),
    )
    def gather_k(table_hbm, idx_hbm, out_hbm, idx_all, idx_chunk, rows_v, sem):
        wid = _worker_id()
        base = wid * PER_W
        pltpu.sync_copy(idx_hbm.at[pl.ds(base, PER_W)], idx_all.at[pl.ds(0, PER_W)])

        @pl.when(wid < TAIL_WORKERS)
        def _():
            pltpu.sync_copy(
                idx_hbm.at[pl.ds(TAIL_BASE + wid * CH, CH)],
                idx_all.at[pl.ds(PER_W, CH)],
            )

        n_my = MAIN_CHUNKS + jnp.where(wid < TAIL_WORKERS, 1, 0)

        def body(i, _):
            g_off, l_off = _chunk_offsets(wid, i)
            _copy_idx_window(idx_all, idx_chunk, l_off)
            pltpu.async_copy(table_hbm.at[idx_chunk], rows_v, sem).wait()
            pltpu.sync_copy(rows_v, out_hbm.at[pl.ds(g_off, CH)])
            return 0

        lax.fori_loop(0, n_my, body, 0)

    return gather_k


def _make_scatter():
    mesh = plsc.VectorSubcoreMesh(core_axis_name="c", subcore_axis_name="s")
    rows_per_sub = N // NS

    @functools.partial(
        pl.kernel,
        mesh=mesh,
        out_type=jax.ShapeDtypeStruct((NC, N, NFC), jnp.float32),
        scratch_types=[
            pltpu.VMEM((PER_W + CH,), jnp.int32),
            pltpu.VMEM((CH,), jnp.int32),
            pltpu.VMEM((CH, NFC), jnp.float32),
            pltpu.VMEM_SHARED((N, NFC), jnp.float32),
        ],
        compiler_params=pltpu.CompilerParams(use_tc_tiling_on_sc=False),
    )
    def scatter_k(rows_hbm, idx_hbm, zeros_hbm, out_hbm, idx_all, idx_chunk, rows_v, acc):
        cid = lax.axis_index("c")
        sid = lax.axis_index("s")
        wid = _worker_id()
        base = wid * PER_W

        pltpu.sync_copy(
            zeros_hbm.at[pl.ds(sid * rows_per_sub, rows_per_sub)],
            acc.at[pl.ds(sid * rows_per_sub, rows_per_sub)],
        )
        pltpu.sync_copy(idx_hbm.at[pl.ds(base, PER_W)], idx_all.at[pl.ds(0, PER_W)])

        @pl.when(wid < TAIL_WORKERS)
        def _():
            pltpu.sync_copy(
                idx_hbm.at[pl.ds(TAIL_BASE + wid * CH, CH)],
                idx_all.at[pl.ds(PER_W, CH)],
            )

        plsc.subcore_barrier()

        n_my = MAIN_CHUNKS + jnp.where(wid < TAIL_WORKERS, 1, 0)

        def body(i, _):
            g_off, l_off = _chunk_offsets(wid, i)
            _copy_idx_window(idx_all, idx_chunk, l_off)
            pltpu.sync_copy(rows_hbm.at[pl.ds(g_off, CH)], rows_v)
            pltpu.sync_copy(rows_v, acc.at[idx_chunk], add=True)
            return 0

        lax.fori_loop(0, n_my, body, 0)

        plsc.subcore_barrier()
        pltpu.sync_copy(
            acc.at[pl.ds(sid * rows_per_sub, rows_per_sub)],
            out_hbm.at[cid, pl.ds(sid * rows_per_sub, rows_per_sub)],
        )

    return scatter_k


def _dot(a, b, out_dtype=jnp.float32):
    return lax.dot_general(
        a, b, (((1,), (0,)), ((), ())), preferred_element_type=out_dtype
    )


PK = 4
EP = E // PK
B_P = B_E // PK
KP = PK * EFC * NFC


def _transform_body(edge_ref, neigh_ref, s_ref, t_ref, w_ref, bt_ref, out_ref):
    e = edge_ref[...]
    n = neigh_ref[...]
    eb = e.astype(jnp.bfloat16)
    nb = n.astype(jnp.bfloat16)
    erep = _dot(eb, s_ref[...]).astype(jnp.bfloat16)
    ntile = _dot(nb, t_ref[...]).astype(jnp.bfloat16)
    p = erep * ntile
    out_ref[...] = _dot(p, w_ref[...]) + _dot(n, bt_ref[...])


def _transform(epack, npack, w, bt, s, t):
    grid = EP // B_P
    return pl.pallas_call(
        _transform_body,
        grid=(grid,),
        in_specs=[
            pl.BlockSpec((B_P, PK * EFC), lambda i: (i, 0)),
            pl.BlockSpec((B_P, PK * NFC), lambda i: (i, 0)),
            pl.BlockSpec((PK * EFC, KP), lambda i: (0, 0)),
            pl.BlockSpec((PK * NFC, KP), lambda i: (0, 0)),
            pl.BlockSpec((KP, PK * NFC), lambda i: (0, 0)),
            pl.BlockSpec((PK * NFC, PK * NFC), lambda i: (0, 0)),
        ],
        out_specs=pl.BlockSpec((B_P, PK * NFC), lambda i: (i, 0)),
        out_shape=jax.ShapeDtypeStruct((EP, PK * NFC), jnp.float32),
        compiler_params=pltpu.CompilerParams(
            dimension_semantics=("arbitrary",),
        ),
    )(epack, npack, s, t, w, bt)


def _combine_body(a_ref, b_ref, out_ref):
    out_ref[...] = a_ref[...] + b_ref[...]


def _combine(partials):
    np_rows = N * NFC // 128
    pp = partials.reshape(2, np_rows, 128)
    a, b = pp[0], pp[1]
    out = pl.pallas_call(
        _combine_body,
        out_shape=jax.ShapeDtypeStruct((np_rows, 128), jnp.float32),
    )(a, b)
    return out.reshape(N, NFC)


@functools.cache
def _sc_calls():
    return _make_gather(), _make_scatter()


def kernel(node_features, edge_features, pair_indices, kernel, bias):
    _gather_call, _scatter_call = _sc_calls()
    idx_dst = pair_indices[:, 0].astype(jnp.int32)
    idx_src = pair_indices[:, 1].astype(jnp.int32)
    w = kernel.reshape(EFC, NFC, NFC).transpose(0, 2, 1).reshape(EFC * NFC, NFC)
    bt = bias.reshape(NFC, NFC).T
    zeros = jnp.zeros((N, NFC), jnp.float32)
    eye4 = jnp.eye(PK, dtype=jnp.float32)
    s = jnp.kron(jnp.eye(EFC, dtype=jnp.float32), jnp.ones((1, NFC), jnp.float32))
    t = jnp.tile(jnp.eye(NFC, dtype=jnp.float32), (1, EFC))
    s4 = jnp.kron(eye4, s).astype(jnp.bfloat16)
    t4 = jnp.kron(eye4, t).astype(jnp.bfloat16)
    w4 = jnp.kron(eye4, w).astype(jnp.bfloat16)
    bt4 = jnp.kron(eye4, bt)

    neigh = _gather_call(node_features, idx_src)
    epack = edge_features.reshape(EP, PK * EFC)
    npack = neigh.reshape(EP, PK * NFC)
    transformed = _transform(epack, npack, w4, bt4, s4, t4)
    partials = _scatter_call(transformed.reshape(E, NFC), idx_dst, zeros)
    return _combine(partials)


# speedup vs baseline: 5.4062x; 1.0615x over previous
import functools

import jax
import jax.numpy as jnp
from jax import lax
from jax.experimental import pallas as pl
from jax.experimental.pallas import tpu as pltpu
from jax.experimental.pallas import tpu_sc as plsc

N = 10000
E = 160000
NFC = 32
EFC = 16

NC = 2
NS = 16
NW = NC * NS

CH = 624
MAIN_CHUNKS = 8
PER_W = MAIN_CHUNKS * CH
TAIL_BASE = PER_W * NW
CHT = 128
TAIL_WORKERS = (E - TAIL_BASE) // CHT

B_E = 640


def _worker_id():
    return lax.axis_index("s") * NC + lax.axis_index("c")


def _make_gather():
    mesh = plsc.VectorSubcoreMesh(core_axis_name="c", subcore_axis_name="s")

    @functools.partial(
        pl.kernel,
        mesh=mesh,
        out_type=jax.ShapeDtypeStruct((E, NFC), jnp.float32),
        scratch_types=[
            pltpu.VMEM((CH,), jnp.int32),
            pltpu.VMEM((CH,), jnp.int32),
            pltpu.VMEM((CH, NFC), jnp.float32),
            pltpu.VMEM((CH, NFC), jnp.float32),
            pltpu.VMEM((CHT,), jnp.int32),
            pltpu.VMEM((CHT, NFC), jnp.float32),
            pltpu.SemaphoreType.DMA,
            pltpu.SemaphoreType.DMA,
            pltpu.SemaphoreType.DMA,
            pltpu.SemaphoreType.DMA,
        ],
        compiler_params=pltpu.CompilerParams(use_tc_tiling_on_sc=False),
    )
    def gather_k(table_hbm, idx_hbm, out_hbm,
                 idx0, idx1, rows0, rows1, idxt, rowst,
                 semi0, semi1, semg0, semg1):
        wid = _worker_id()
        base = wid * PER_W
        idxb = [idx0, idx1]
        rowsb = [rows0, rows1]
        semi = [semi0, semi1]
        semg = [semg0, semg1]

        ci = [None] * MAIN_CHUNKS
        cg = [None] * MAIN_CHUNKS
        ci[0] = pltpu.async_copy(idx_hbm.at[pl.ds(base, CH)], idxb[0], semi[0])
        for i in range(MAIN_CHUNKS):
            b = i % 2
            ci[i].wait()
            cg[i] = pltpu.async_copy(table_hbm.at[idxb[b]], rowsb[b], semg[b])
            if i + 1 < MAIN_CHUNKS:
                ci[i + 1] = pltpu.async_copy(
                    idx_hbm.at[pl.ds(base + (i + 1) * CH, CH)],
                    idxb[(i + 1) % 2], semi[(i + 1) % 2])
            if i >= 1:
                cg[i - 1].wait()
                pltpu.sync_copy(rowsb[(i - 1) % 2],
                                out_hbm.at[pl.ds(base + (i - 1) * CH, CH)])
        cg[MAIN_CHUNKS - 1].wait()
        last = MAIN_CHUNKS - 1
        pltpu.sync_copy(rowsb[last % 2],
                        out_hbm.at[pl.ds(base + last * CH, CH)])

        @pl.when(wid < TAIL_WORKERS)
        def _():
            toff = TAIL_BASE + wid * CHT
            pltpu.sync_copy(idx_hbm.at[pl.ds(toff, CHT)], idxt)
            pltpu.async_copy(table_hbm.at[idxt], rowst, semg0).wait()
            pltpu.sync_copy(rowst, out_hbm.at[pl.ds(toff, CHT)])

    return gather_k


def _make_scatter():
    mesh = plsc.VectorSubcoreMesh(core_axis_name="c", subcore_axis_name="s")
    rows_per_sub = N // NS

    @functools.partial(
        pl.kernel,
        mesh=mesh,
        out_type=jax.ShapeDtypeStruct((NC, N, NFC), jnp.float32),
        scratch_types=[
            pltpu.VMEM((CH,), jnp.int32),
            pltpu.VMEM((CH,), jnp.int32),
            pltpu.VMEM((CH, NFC), jnp.float32),
            pltpu.VMEM((CH, NFC), jnp.float32),
            pltpu.VMEM((CHT,), jnp.int32),
            pltpu.VMEM((CHT, NFC), jnp.float32),
            pltpu.VMEM_SHARED((N, NFC), jnp.float32),
            pltpu.SemaphoreType.DMA,
            pltpu.SemaphoreType.DMA,
            pltpu.SemaphoreType.DMA,
            pltpu.SemaphoreType.DMA,
        ],
        compiler_params=pltpu.CompilerParams(use_tc_tiling_on_sc=False),
    )
    def scatter_k(rows_hbm, idx_hbm, zeros_hbm, out_hbm,
                  idx0, idx1, rows0, rows1, idxt, rowst, acc,
                  semi0, semi1, semr0, semr1):
        cid = lax.axis_index("c")
        sid = lax.axis_index("s")
        wid = _worker_id()
        base = wid * PER_W
        idxb = [idx0, idx1]
        rowsb = [rows0, rows1]
        semi = [semi0, semi1]
        semr = [semr0, semr1]

        pltpu.sync_copy(
            zeros_hbm.at[pl.ds(sid * rows_per_sub, rows_per_sub)],
            acc.at[pl.ds(sid * rows_per_sub, rows_per_sub)],
        )
        plsc.subcore_barrier()

        ci = [None] * MAIN_CHUNKS
        cr = [None] * MAIN_CHUNKS
        ci[0] = pltpu.async_copy(idx_hbm.at[pl.ds(base, CH)], idxb[0], semi[0])
        cr[0] = pltpu.async_copy(rows_hbm.at[pl.ds(base, CH)], rowsb[0], semr[0])
        for i in range(MAIN_CHUNKS):
            b = i % 2
            if i + 1 < MAIN_CHUNKS:
                nb = (i + 1) % 2
                off = base + (i + 1) * CH
                ci[i + 1] = pltpu.async_copy(
                    idx_hbm.at[pl.ds(off, CH)], idxb[nb], semi[nb])
                cr[i + 1] = pltpu.async_copy(
                    rows_hbm.at[pl.ds(off, CH)], rowsb[nb], semr[nb])
            ci[i].wait()
            cr[i].wait()
            pltpu.sync_copy(rowsb[b], acc.at[idxb[b]], add=True)

        @pl.when(wid < TAIL_WORKERS)
        def _():
            toff = TAIL_BASE + wid * CHT
            pltpu.sync_copy(idx_hbm.at[pl.ds(toff, CHT)], idxt)
            pltpu.sync_copy(rows_hbm.at[pl.ds(toff, CHT)], rowst)
            pltpu.sync_copy(rowst, acc.at[idxt], add=True)

        plsc.subcore_barrier()
        pltpu.sync_copy(
            acc.at[pl.ds(sid * rows_per_sub, rows_per_sub)],
            out_hbm.at[cid, pl.ds(sid * rows_per_sub, rows_per_sub)],
        )

    return scatter_k


def _dot(a, b, out_dtype=jnp.float32):
    return lax.dot_general(
        a, b, (((1,), (0,)), ((), ())), preferred_element_type=out_dtype
    )


PK = 4
EP = E // PK
B_P = B_E // PK
KP = PK * EFC * NFC


def _transform_body(edge_ref, neigh_ref, s_ref, t_ref, w_ref, bt_ref, out_ref):
    e = edge_ref[...]
    n = neigh_ref[...]
    eb = e.astype(jnp.bfloat16)
    nb = n.astype(jnp.bfloat16)
    erep = _dot(eb, s_ref[...]).astype(jnp.bfloat16)
    ntile = _dot(nb, t_ref[...]).astype(jnp.bfloat16)
    p = erep * ntile
    out_ref[...] = _dot(p, w_ref[...]) + _dot(n, bt_ref[...])


def _transform(epack, npack, w, bt, s, t):
    grid = EP // B_P
    return pl.pallas_call(
        _transform_body,
        grid=(grid,),
        in_specs=[
            pl.BlockSpec((B_P, PK * EFC), lambda i: (i, 0)),
            pl.BlockSpec((B_P, PK * NFC), lambda i: (i, 0)),
            pl.BlockSpec((PK * EFC, KP), lambda i: (0, 0)),
            pl.BlockSpec((PK * NFC, KP), lambda i: (0, 0)),
            pl.BlockSpec((KP, PK * NFC), lambda i: (0, 0)),
            pl.BlockSpec((PK * NFC, PK * NFC), lambda i: (0, 0)),
        ],
        out_specs=pl.BlockSpec((B_P, PK * NFC), lambda i: (i, 0)),
        out_shape=jax.ShapeDtypeStruct((EP, PK * NFC), jnp.float32),
        compiler_params=pltpu.CompilerParams(
            dimension_semantics=("arbitrary",),
        ),
    )(epack, npack, s, t, w, bt)


def _combine_body(a_ref, b_ref, out_ref):
    out_ref[...] = a_ref[...] + b_ref[...]


def _combine(partials):
    np_rows = N * NFC // 128
    pp = partials.reshape(2, np_rows, 128)
    a, b = pp[0], pp[1]
    out = pl.pallas_call(
        _combine_body,
        out_shape=jax.ShapeDtypeStruct((np_rows, 128), jnp.float32),
    )(a, b)
    return out.reshape(N, NFC)


@functools.cache
def _sc_calls():
    return _make_gather(), _make_scatter()


def kernel(node_features, edge_features, pair_indices, kernel, bias):
    _gather_call, _scatter_call = _sc_calls()
    idx_dst = pair_indices[:, 0].astype(jnp.int32)
    idx_src = pair_indices[:, 1].astype(jnp.int32)
    w = kernel.reshape(EFC, NFC, NFC).transpose(0, 2, 1).reshape(EFC * NFC, NFC)
    bt = bias.reshape(NFC, NFC).T
    zeros = jnp.zeros((N, NFC), jnp.float32)
    eye4 = jnp.eye(PK, dtype=jnp.float32)
    s = jnp.kron(jnp.eye(EFC, dtype=jnp.float32), jnp.ones((1, NFC), jnp.float32))
    t = jnp.tile(jnp.eye(NFC, dtype=jnp.float32), (1, EFC))
    s4 = jnp.kron(eye4, s).astype(jnp.bfloat16)
    t4 = jnp.kron(eye4, t).astype(jnp.bfloat16)
    w4 = jnp.kron(eye4, w).astype(jnp.bfloat16)
    bt4 = jnp.kron(eye4, bt)

    neigh = _gather_call(node_features, idx_src)
    epack = edge_features.reshape(EP, PK * EFC)
    npack = neigh.reshape(EP, PK * NFC)
    transformed = _transform(epack, npack, w4, bt4, s4, t4)
    partials = _scatter_call(transformed.reshape(E, NFC), idx_dst, zeros)
    return _combine(partials)


# speedup vs baseline: 6.1087x; 1.1299x over previous
import functools

import jax
import jax.numpy as jnp
from jax import lax
from jax.experimental import pallas as pl
from jax.experimental.pallas import tpu as pltpu
from jax.experimental.pallas import tpu_sc as plsc

N = 10000
E = 160000
NFC = 32
EFC = 16

NC = 2
NS = 16
NW = NC * NS

CH = 624
MAIN_CHUNKS = 8
PER_W = MAIN_CHUNKS * CH
TAIL_BASE = PER_W * NW
CHT = 128
TAIL_WORKERS = (E - TAIL_BASE) // CHT

B_E = 3200


def _worker_id():
    return lax.axis_index("s") * NC + lax.axis_index("c")


def _make_gather():
    mesh = plsc.VectorSubcoreMesh(core_axis_name="c", subcore_axis_name="s")

    @functools.partial(
        pl.kernel,
        mesh=mesh,
        out_type=jax.ShapeDtypeStruct((E, NFC), jnp.float32),
        scratch_types=[
            pltpu.VMEM((CH,), jnp.int32),
            pltpu.VMEM((CH,), jnp.int32),
            pltpu.VMEM((CH, NFC), jnp.float32),
            pltpu.VMEM((CH, NFC), jnp.float32),
            pltpu.VMEM((CHT,), jnp.int32),
            pltpu.VMEM((CHT, NFC), jnp.float32),
            pltpu.SemaphoreType.DMA,
            pltpu.SemaphoreType.DMA,
            pltpu.SemaphoreType.DMA,
            pltpu.SemaphoreType.DMA,
        ],
        compiler_params=pltpu.CompilerParams(use_tc_tiling_on_sc=False),
    )
    def gather_k(table_hbm, idx_hbm, out_hbm,
                 idx0, idx1, rows0, rows1, idxt, rowst,
                 semi0, semi1, semg0, semg1):
        wid = _worker_id()
        base = wid * PER_W
        idxb = [idx0, idx1]
        rowsb = [rows0, rows1]
        semi = [semi0, semi1]
        semg = [semg0, semg1]

        ci = [None] * MAIN_CHUNKS
        cg = [None] * MAIN_CHUNKS
        ci[0] = pltpu.async_copy(idx_hbm.at[pl.ds(base, CH)], idxb[0], semi[0])
        for i in range(MAIN_CHUNKS):
            b = i % 2
            ci[i].wait()
            cg[i] = pltpu.async_copy(table_hbm.at[idxb[b]], rowsb[b], semg[b])
            if i + 1 < MAIN_CHUNKS:
                ci[i + 1] = pltpu.async_copy(
                    idx_hbm.at[pl.ds(base + (i + 1) * CH, CH)],
                    idxb[(i + 1) % 2], semi[(i + 1) % 2])
            if i >= 1:
                cg[i - 1].wait()
                pltpu.sync_copy(rowsb[(i - 1) % 2],
                                out_hbm.at[pl.ds(base + (i - 1) * CH, CH)])
        cg[MAIN_CHUNKS - 1].wait()
        last = MAIN_CHUNKS - 1
        pltpu.sync_copy(rowsb[last % 2],
                        out_hbm.at[pl.ds(base + last * CH, CH)])

        @pl.when(wid < TAIL_WORKERS)
        def _():
            toff = TAIL_BASE + wid * CHT
            pltpu.sync_copy(idx_hbm.at[pl.ds(toff, CHT)], idxt)
            pltpu.async_copy(table_hbm.at[idxt], rowst, semg0).wait()
            pltpu.sync_copy(rowst, out_hbm.at[pl.ds(toff, CHT)])

    return gather_k


def _make_scatter():
    mesh = plsc.VectorSubcoreMesh(core_axis_name="c", subcore_axis_name="s")
    rows_per_sub = N // NS

    @functools.partial(
        pl.kernel,
        mesh=mesh,
        out_type=jax.ShapeDtypeStruct((NC, N, NFC), jnp.float32),
        scratch_types=[
            pltpu.VMEM((CH,), jnp.int32),
            pltpu.VMEM((CH,), jnp.int32),
            pltpu.VMEM((CH, NFC), jnp.float32),
            pltpu.VMEM((CH, NFC), jnp.float32),
            pltpu.VMEM((CHT,), jnp.int32),
            pltpu.VMEM((CHT, NFC), jnp.float32),
            pltpu.VMEM_SHARED((N, NFC), jnp.float32),
            pltpu.SemaphoreType.DMA,
            pltpu.SemaphoreType.DMA,
            pltpu.SemaphoreType.DMA,
            pltpu.SemaphoreType.DMA,
        ],
        compiler_params=pltpu.CompilerParams(use_tc_tiling_on_sc=False),
    )
    def scatter_k(rows_hbm, idx_hbm, zeros_hbm, out_hbm,
                  idx0, idx1, rows0, rows1, idxt, rowst, acc,
                  semi0, semi1, semr0, semr1):
        cid = lax.axis_index("c")
        sid = lax.axis_index("s")
        wid = _worker_id()
        base = wid * PER_W
        idxb = [idx0, idx1]
        rowsb = [rows0, rows1]
        semi = [semi0, semi1]
        semr = [semr0, semr1]

        pltpu.sync_copy(
            zeros_hbm.at[pl.ds(sid * rows_per_sub, rows_per_sub)],
            acc.at[pl.ds(sid * rows_per_sub, rows_per_sub)],
        )
        plsc.subcore_barrier()

        ci = [None] * MAIN_CHUNKS
        cr = [None] * MAIN_CHUNKS
        ci[0] = pltpu.async_copy(idx_hbm.at[pl.ds(base, CH)], idxb[0], semi[0])
        cr[0] = pltpu.async_copy(rows_hbm.at[pl.ds(base, CH)], rowsb[0], semr[0])
        for i in range(MAIN_CHUNKS):
            b = i % 2
            if i + 1 < MAIN_CHUNKS:
                nb = (i + 1) % 2
                off = base + (i + 1) * CH
                ci[i + 1] = pltpu.async_copy(
                    idx_hbm.at[pl.ds(off, CH)], idxb[nb], semi[nb])
                cr[i + 1] = pltpu.async_copy(
                    rows_hbm.at[pl.ds(off, CH)], rowsb[nb], semr[nb])
            ci[i].wait()
            cr[i].wait()
            pltpu.sync_copy(rowsb[b], acc.at[idxb[b]], add=True)

        @pl.when(wid < TAIL_WORKERS)
        def _():
            toff = TAIL_BASE + wid * CHT
            pltpu.sync_copy(idx_hbm.at[pl.ds(toff, CHT)], idxt)
            pltpu.sync_copy(rows_hbm.at[pl.ds(toff, CHT)], rowst)
            pltpu.sync_copy(rowst, acc.at[idxt], add=True)

        plsc.subcore_barrier()
        pltpu.sync_copy(
            acc.at[pl.ds(sid * rows_per_sub, rows_per_sub)],
            out_hbm.at[cid, pl.ds(sid * rows_per_sub, rows_per_sub)],
        )

    return scatter_k


def _dot(a, b, out_dtype=jnp.float32):
    return lax.dot_general(
        a, b, (((1,), (0,)), ((), ())), preferred_element_type=out_dtype
    )


PK = 4
EP = E // PK
B_P = B_E // PK
KP = PK * EFC * NFC


def _transform_body(edge_ref, neigh_ref, s_ref, t_ref, w_ref, bt_ref, out_ref):
    e = edge_ref[...]
    n = neigh_ref[...]
    eb = e.astype(jnp.bfloat16)
    nb = n.astype(jnp.bfloat16)
    erep = _dot(eb, s_ref[...]).astype(jnp.bfloat16)
    ntile = _dot(nb, t_ref[...]).astype(jnp.bfloat16)
    p = erep * ntile
    out_ref[...] = _dot(p, w_ref[...]) + _dot(nb, bt_ref[...])


def _transform(epack, npack, w, bt, s, t):
    grid = EP // B_P
    return pl.pallas_call(
        _transform_body,
        grid=(grid,),
        in_specs=[
            pl.BlockSpec((B_P, PK * EFC), lambda i: (i, 0)),
            pl.BlockSpec((B_P, PK * NFC), lambda i: (i, 0)),
            pl.BlockSpec((PK * EFC, KP), lambda i: (0, 0)),
            pl.BlockSpec((PK * NFC, KP), lambda i: (0, 0)),
            pl.BlockSpec((KP, PK * NFC), lambda i: (0, 0)),
            pl.BlockSpec((PK * NFC, PK * NFC), lambda i: (0, 0)),
        ],
        out_specs=pl.BlockSpec((B_P, PK * NFC), lambda i: (i, 0)),
        out_shape=jax.ShapeDtypeStruct((EP, PK * NFC), jnp.float32),
        compiler_params=pltpu.CompilerParams(
            dimension_semantics=("arbitrary",),
        ),
    )(epack, npack, s, t, w, bt)


def _combine_body(a_ref, b_ref, out_ref):
    out_ref[...] = a_ref[...] + b_ref[...]


def _combine(partials):
    np_rows = N * NFC // 128
    pp = partials.reshape(2, np_rows, 128)
    a, b = pp[0], pp[1]
    out = pl.pallas_call(
        _combine_body,
        out_shape=jax.ShapeDtypeStruct((np_rows, 128), jnp.float32),
    )(a, b)
    return out.reshape(N, NFC)


@functools.cache
def _sc_calls():
    return _make_gather(), _make_scatter()


def kernel(node_features, edge_features, pair_indices, kernel, bias):
    _gather_call, _scatter_call = _sc_calls()
    idx_dst = pair_indices[:, 0].astype(jnp.int32)
    idx_src = pair_indices[:, 1].astype(jnp.int32)
    w = kernel.reshape(EFC, NFC, NFC).transpose(0, 2, 1).reshape(EFC * NFC, NFC)
    bt = bias.reshape(NFC, NFC).T
    zeros = jnp.zeros((N, NFC), jnp.float32)
    eye4 = jnp.eye(PK, dtype=jnp.float32)
    s = jnp.kron(jnp.eye(EFC, dtype=jnp.float32), jnp.ones((1, NFC), jnp.float32))
    t = jnp.tile(jnp.eye(NFC, dtype=jnp.float32), (1, EFC))
    s4 = jnp.kron(eye4, s).astype(jnp.bfloat16)
    t4 = jnp.kron(eye4, t).astype(jnp.bfloat16)
    w4 = jnp.kron(eye4, w).astype(jnp.bfloat16)
    bt4 = jnp.kron(eye4, bt).astype(jnp.bfloat16)

    neigh = _gather_call(node_features, idx_src)
    epack = edge_features.reshape(EP, PK * EFC)
    npack = neigh.reshape(EP, PK * NFC)
    transformed = _transform(epack, npack, w4, bt4, s4, t4)
    partials = _scatter_call(transformed.reshape(E, NFC), idx_dst, zeros)
    return _combine(partials)
